# baseline (speedup 1.0000x reference)
import functools

import jax
import jax.numpy as jnp
from jax import lax
from jax.experimental import pallas as pl
from jax.experimental.pallas import tpu as pltpu
from jax.experimental.pallas import tpu_sc as plsc

N_NODES = 10000
N_EDGES = 320000
ND = 128
ED = 16
H = 64

NC = 2
NS = 16
L = 16
NW = NC * NS
CH = 128
NCHUNK = 80
EPW = CH * NCHUNK
EP = EPW * NW
N_PAD = 10240
RPT = N_PAD // NS
ZC = 128


def _gather_body(s_hbm, d_hbm, sidx_hbm, didx_hbm, gs_hbm, gd_hbm,
                 sidx_v, didx_v, rs0, rd0, rs1, rd1,
                 sem_g0, sem_g1, sem_w0, sem_w1):
    wid = lax.axis_index("c") * NS + lax.axis_index("s")
    pltpu.sync_copy(sidx_hbm.at[wid], sidx_v)
    pltpu.sync_copy(didx_hbm.at[wid], didx_v)
    rs = (rs0, rs1)
    rd = (rd0, rd1)
    sg = (sem_g0, sem_g1)
    sw = (sem_w0, sem_w1)

    def g_start(j, b):
        pltpu.async_copy(s_hbm.at[sidx_v.at[j]], rs[b], sg[b])
        pltpu.async_copy(d_hbm.at[didx_v.at[j]], rd[b], sg[b])

    def g_wait(b):
        pltpu.make_async_copy(s_hbm.at[sidx_v.at[0]], rs[b], sg[b]).wait()
        pltpu.make_async_copy(d_hbm.at[didx_v.at[0]], rd[b], sg[b]).wait()

    def w_start(j, b):
        pltpu.async_copy(rs[b], gs_hbm.at[wid, j], sw[b])
        pltpu.async_copy(rd[b], gd_hbm.at[wid, j], sw[b])

    def w_wait(b):
        pltpu.make_async_copy(rs[b], gs_hbm.at[wid, 0], sw[b]).wait()
        pltpu.make_async_copy(rd[b], gd_hbm.at[wid, 0], sw[b]).wait()

    g_start(0, 0)

    def outer(i, carry):
        for b in range(2):
            j = 2 * i + b
            nb = 1 - b

            @pl.when(j + 1 < NCHUNK)
            def _start_next():
                @pl.when(j >= 1)
                def _drain_write():
                    w_wait(nb)
                g_start(j + 1, nb)

            g_wait(b)
            w_start(j, b)
        return carry

    lax.fori_loop(0, NCHUNK // 2, outer, 0)
    w_wait(0)
    w_wait(1)


def _scatter_sum_body(msg_hbm, didx_hbm, out_hbm,
                      didx_v, m0, m1, agg_sh,
                      sem_l0, sem_l1, sem_s0, sem_s1):
    cid = lax.axis_index("c")
    sid = lax.axis_index("s")
    wid = cid * NS + sid
    m = (m0, m1)
    sl_ = (sem_l0, sem_l1)
    ss = (sem_s0, sem_s1)
    NB = 2

    def zrow(r, c2):
        for cc in range(ND // L):
            m0[r, pl.ds(cc * L, L)] = jnp.zeros((L,), jnp.float32)
        return c2

    lax.fori_loop(0, ZC, zrow, 0)
    for k in range(RPT // ZC):
        pltpu.sync_copy(m0, agg_sh.at[pl.ds(sid * RPT + k * ZC, ZC)])
    pltpu.sync_copy(didx_hbm.at[wid], didx_v)
    plsc.subcore_barrier()

    def l_start(j, b):
        pltpu.async_copy(msg_hbm.at[wid, j], m[b], sl_[b])

    def l_wait(b):
        pltpu.make_async_copy(msg_hbm.at[wid, 0], m[b], sl_[b]).wait()

    def s_start(j, b):
        pltpu.async_copy(m[b], agg_sh.at[didx_v.at[j]], ss[b], add=True)

    def s_wait(b):
        pltpu.make_async_copy(m[b], agg_sh.at[didx_v.at[0]], ss[b]).wait()

    for b in range(NB):
        l_start(b, b)

    def outer(i, carry):
        for b in range(NB):
            j = NB * i + b
            l_wait(b)
            s_start(j, b)

            @pl.when(j + NB < NCHUNK)
            def _start_next():
                s_wait(b)
                l_start(j + NB, b)
        return carry

    lax.fori_loop(0, NCHUNK // NB, outer, 0)
    for b in range(NB):
        s_wait(b)
    plsc.subcore_barrier()
    pltpu.sync_copy(agg_sh.at[pl.ds(sid * RPT, RPT)],
                    out_hbm.at[cid, pl.ds(sid * RPT, RPT)])


@functools.lru_cache(maxsize=1)
def _sc_kernels():
    mesh = plsc.VectorSubcoreMesh(
        core_axis_name="c", subcore_axis_name="s", num_cores=NC, num_subcores=NS)
    gather = pl.kernel(
        _gather_body,
        out_type=[jax.ShapeDtypeStruct((NW, NCHUNK, CH, ND), jnp.float32),
                  jax.ShapeDtypeStruct((NW, NCHUNK, CH, ND), jnp.float32)],
        mesh=mesh,
        scratch_types=[
            pltpu.VMEM((NCHUNK, CH), jnp.int32),
            pltpu.VMEM((NCHUNK, CH), jnp.int32),
            pltpu.VMEM((CH, ND), jnp.float32),
            pltpu.VMEM((CH, ND), jnp.float32),
            pltpu.VMEM((CH, ND), jnp.float32),
            pltpu.VMEM((CH, ND), jnp.float32),
            pltpu.SemaphoreType.DMA,
            pltpu.SemaphoreType.DMA,
            pltpu.SemaphoreType.DMA,
            pltpu.SemaphoreType.DMA,
        ],
    )
    scatter_sum = pl.kernel(
        _scatter_sum_body,
        out_type=jax.ShapeDtypeStruct((NC, N_PAD, ND), jnp.float32),
        mesh=mesh,
        scratch_types=[
            pltpu.VMEM((NCHUNK, CH), jnp.int32),
            pltpu.VMEM((CH, ND), jnp.float32),
            pltpu.VMEM((CH, ND), jnp.float32),
            pltpu.VMEM_SHARED((N_PAD, ND), jnp.float32),
            pltpu.SemaphoreType.DMA,
            pltpu.SemaphoreType.DMA,
            pltpu.SemaphoreType.DMA,
            pltpu.SemaphoreType.DMA,
        ],
    )
    return gather, scatter_sum


def _proj_body(x_ref, ws_ref, wd_ref, s_ref, d_ref):
    xv = x_ref[...]
    s_ref[...] = jnp.dot(xv, ws_ref[...], preferred_element_type=jnp.float32)
    d_ref[...] = jnp.dot(xv, wd_ref[...], preferred_element_type=jnp.float32)


def _msg_body(gs_ref, gd_ref, ea_ref, we1_ref, b1_ref, wg2_ref, bg2_ref,
              wo2_ref, bo2_ref, msg_ref):
    pre = (gs_ref[...] + gd_ref[...]
           + jnp.dot(ea_ref[...], we1_ref[...], preferred_element_type=jnp.float32)
           + b1_ref[...])
    hg = jax.nn.silu(pre[:, :H])
    ho = jax.nn.silu(pre[:, H:])
    gate = jax.nn.sigmoid(
        jnp.dot(hg, wg2_ref[...], preferred_element_type=jnp.float32) + bg2_ref[...])
    val = jax.nn.silu(
        jnp.dot(ho, wo2_ref[...], preferred_element_type=jnp.float32) + bo2_ref[...])
    msg_ref[...] = gate * val


def _final_body(x_ref, p_ref, wl_ref, bl_ref, out_ref):
    agg = p_ref[0] + p_ref[1]
    out_ref[...] = (x_ref[...]
                    + jnp.dot(agg, wl_ref[...], preferred_element_type=jnp.float32)
                    + bl_ref[...])


def kernel(x, edge_index, edge_attr, Wg1, bg1, Wg2, bg2, Wo1, bo1, Wo2, bo2, Wl, bl):
    f32 = jnp.float32
    w_src = jnp.concatenate([Wg1[:ND], Wo1[:ND]], axis=1)
    w_dst = jnp.concatenate([Wg1[ND:2 * ND], Wo1[ND:2 * ND]], axis=1)
    w_e = jnp.concatenate([Wg1[2 * ND:], Wo1[2 * ND:]], axis=1)
    b1 = jnp.concatenate([bg1, bo1]).reshape(1, 2 * H)
    bg2r = bg2.reshape(1, ND)
    bo2r = bo2.reshape(1, ND)
    blr = bl.reshape(1, ND)

    npad = EP - N_EDGES
    pad_gather = (jnp.arange(npad, dtype=jnp.int32) % N_NODES)
    pad_scatter = N_NODES + (jnp.arange(npad, dtype=jnp.int32) % (N_PAD - N_NODES))
    src = jnp.concatenate([edge_index[0], pad_gather]).reshape(NW, NCHUNK, CH)
    dst_g = jnp.concatenate([edge_index[1], pad_gather]).reshape(NW, NCHUNK, CH)
    dst_s = jnp.concatenate([edge_index[1], pad_scatter]).reshape(NW, NCHUNK, CH)
    ea_pad = jnp.concatenate(
        [edge_attr, jnp.zeros((npad, ED), dtype=f32)], axis=0)

    S, D = pl.pallas_call(
        _proj_body,
        out_shape=[jax.ShapeDtypeStruct((N_NODES, ND), f32),
                   jax.ShapeDtypeStruct((N_NODES, ND), f32)],
    )(x, w_src, w_dst)

    gather, scatter_sum = _sc_kernels()

    GS, GD = gather(S, D, src, dst_g)
    GS = GS.reshape(EP, ND)
    GD = GD.reshape(EP, ND)

    BE = 4096
    grid = EP // BE
    msg = pl.pallas_call(
        _msg_body,
        grid=(grid,),
        in_specs=[
            pl.BlockSpec((BE, ND), lambda i: (i, 0)),
            pl.BlockSpec((BE, ND), lambda i: (i, 0)),
            pl.BlockSpec((BE, ED), lambda i: (i, 0)),
            pl.BlockSpec((ED, 2 * H), lambda i: (0, 0)),
            pl.BlockSpec((1, 2 * H), lambda i: (0, 0)),
            pl.BlockSpec((H, ND), lambda i: (0, 0)),
            pl.BlockSpec((1, ND), lambda i: (0, 0)),
            pl.BlockSpec((H, ND), lambda i: (0, 0)),
            pl.BlockSpec((1, ND), lambda i: (0, 0)),
        ],
        out_specs=pl.BlockSpec((BE, ND), lambda i: (i, 0)),
        out_shape=jax.ShapeDtypeStruct((EP, ND), f32),
    )(GS, GD, ea_pad, w_e, b1, Wg2, bg2r, Wo2, bo2r)

    partials = scatter_sum(msg.reshape(NW, NCHUNK, CH, ND), dst_s)

    BN = 2000
    out = pl.pallas_call(
        _final_body,
        grid=(N_NODES // BN,),
        in_specs=[
            pl.BlockSpec((BN, ND), lambda i: (i, 0)),
            pl.BlockSpec((NC, BN, ND), lambda i: (0, i, 0)),
            pl.BlockSpec((ND, ND), lambda i: (0, 0)),
            pl.BlockSpec((1, ND), lambda i: (0, 0)),
        ],
        out_specs=pl.BlockSpec((BN, ND), lambda i: (i, 0)),
        out_shape=jax.ShapeDtypeStruct((N_NODES, ND), f32),
    )(x, partials, Wl, blr)
    return out


# speedup vs baseline: 1.1202x; 1.1202x over previous
import functools

import jax
import jax.numpy as jnp
from jax import lax
from jax.experimental import pallas as pl
from jax.experimental.pallas import tpu as pltpu
from jax.experimental.pallas import tpu_sc as plsc

N_NODES = 10000
N_EDGES = 320000
ND = 128
ED = 16
H = 64

NC = 2
NS = 16
L = 16
NW = NC * NS
CH = 128
NCHUNK = 80
EPW = CH * NCHUNK
EP = EPW * NW
N_PAD = 10240
RPT = N_PAD // NS
ZC = 128


def _gather_add_body(s_hbm, d_hbm, sidx_hbm, didx_hbm, g_hbm,
                     sidx_v, didx_v, rs0, rd0, rs1, rd1,
                     sem_g0, sem_g1, sem_w0, sem_w1):
    wid = lax.axis_index("c") * NS + lax.axis_index("s")
    pltpu.sync_copy(sidx_hbm.at[wid], sidx_v)
    pltpu.sync_copy(didx_hbm.at[wid], didx_v)
    rs = (rs0, rs1)
    rd = (rd0, rd1)
    sg = (sem_g0, sem_g1)
    sw = (sem_w0, sem_w1)

    def g_start(j, b):
        pltpu.async_copy(s_hbm.at[sidx_v.at[j]], rs[b], sg[b])
        pltpu.async_copy(d_hbm.at[didx_v.at[j]], rd[b], sg[b])

    def g_wait(b):
        pltpu.make_async_copy(s_hbm.at[sidx_v.at[0]], rs[b], sg[b]).wait()
        pltpu.make_async_copy(d_hbm.at[didx_v.at[0]], rd[b], sg[b]).wait()

    def w_start(j, b):
        pltpu.async_copy(rs[b], g_hbm.at[wid, j], sw[b])

    def w_wait(b):
        pltpu.make_async_copy(rs[b], g_hbm.at[wid, 0], sw[b]).wait()

    g_start(0, 0)

    def outer(i, carry):
        for b in range(2):
            j = 2 * i + b
            nb = 1 - b

            @pl.when(j + 1 < NCHUNK)
            def _start_next():
                @pl.when(j >= 1)
                def _drain_write():
                    w_wait(nb)
                g_start(j + 1, nb)

            g_wait(b)

            def addrow(r, c2):
                for cc in range(ND // L):
                    sl = pl.ds(cc * L, L)
                    plsc.addupdate(rs[b].at[r, sl], rd[b][r, sl])
                return c2

            lax.fori_loop(0, CH, addrow, 0)
            w_start(j, b)
        return carry

    lax.fori_loop(0, NCHUNK // 2, outer, 0)
    w_wait(0)
    w_wait(1)


def _scatter_sum_body(msg_hbm, didx_hbm, out_hbm,
                      didx_v, m0, m1, agg_sh,
                      sem_l0, sem_l1, sem_s0, sem_s1):
    cid = lax.axis_index("c")
    sid = lax.axis_index("s")
    wid = cid * NS + sid
    m = (m0, m1)
    sl_ = (sem_l0, sem_l1)
    ss = (sem_s0, sem_s1)
    NB = 2

    def zrow(r, c2):
        for cc in range(ND // L):
            m0[r, pl.ds(cc * L, L)] = jnp.zeros((L,), jnp.float32)
        return c2

    lax.fori_loop(0, ZC, zrow, 0)
    for k in range(RPT // ZC):
        pltpu.sync_copy(m0, agg_sh.at[pl.ds(sid * RPT + k * ZC, ZC)])
    pltpu.sync_copy(didx_hbm.at[wid], didx_v)
    plsc.subcore_barrier()

    def l_start(j, b):
        pltpu.async_copy(msg_hbm.at[wid, j], m[b], sl_[b])

    def l_wait(b):
        pltpu.make_async_copy(msg_hbm.at[wid, 0], m[b], sl_[b]).wait()

    def s_start(j, b):
        pltpu.async_copy(m[b], agg_sh.at[didx_v.at[j]], ss[b], add=True)

    def s_wait(b):
        pltpu.make_async_copy(m[b], agg_sh.at[didx_v.at[0]], ss[b]).wait()

    for b in range(NB):
        l_start(b, b)

    def outer(i, carry):
        for b in range(NB):
            j = NB * i + b
            l_wait(b)
            s_start(j, b)

            @pl.when(j + NB < NCHUNK)
            def _start_next():
                s_wait(b)
                l_start(j + NB, b)
        return carry

    lax.fori_loop(0, NCHUNK // NB, outer, 0)
    for b in range(NB):
        s_wait(b)
    plsc.subcore_barrier()
    pltpu.sync_copy(agg_sh.at[pl.ds(sid * RPT, RPT)],
                    out_hbm.at[cid, pl.ds(sid * RPT, RPT)])


@functools.lru_cache(maxsize=1)
def _sc_kernels():
    mesh = plsc.VectorSubcoreMesh(
        core_axis_name="c", subcore_axis_name="s", num_cores=NC, num_subcores=NS)
    gather_add = pl.kernel(
        _gather_add_body,
        out_type=jax.ShapeDtypeStruct((NW, NCHUNK, CH, ND), jnp.float32),
        mesh=mesh,
        scratch_types=[
            pltpu.VMEM((NCHUNK, CH), jnp.int32),
            pltpu.VMEM((NCHUNK, CH), jnp.int32),
            pltpu.VMEM((CH, ND), jnp.float32),
            pltpu.VMEM((CH, ND), jnp.float32),
            pltpu.VMEM((CH, ND), jnp.float32),
            pltpu.VMEM((CH, ND), jnp.float32),
            pltpu.SemaphoreType.DMA,
            pltpu.SemaphoreType.DMA,
            pltpu.SemaphoreType.DMA,
            pltpu.SemaphoreType.DMA,
        ],
    )
    scatter_sum = pl.kernel(
        _scatter_sum_body,
        out_type=jax.ShapeDtypeStruct((NC, N_PAD, ND), jnp.float32),
        mesh=mesh,
        scratch_types=[
            pltpu.VMEM((NCHUNK, CH), jnp.int32),
            pltpu.VMEM((CH, ND), jnp.float32),
            pltpu.VMEM((CH, ND), jnp.float32),
            pltpu.VMEM_SHARED((N_PAD, ND), jnp.float32),
            pltpu.SemaphoreType.DMA,
            pltpu.SemaphoreType.DMA,
            pltpu.SemaphoreType.DMA,
            pltpu.SemaphoreType.DMA,
        ],
    )
    return gather_add, scatter_sum


def _proj_body(x_ref, ws_ref, wd_ref, s_ref, d_ref):
    xv = x_ref[...]
    s_ref[...] = jnp.dot(xv, ws_ref[...], preferred_element_type=jnp.float32)
    d_ref[...] = jnp.dot(xv, wd_ref[...], preferred_element_type=jnp.float32)


def _msg_body(g_ref, ea_ref, we1_ref, b1_ref, wg2_ref, bg2_ref,
              wo2_ref, bo2_ref, msg_ref):
    pre = (g_ref[...]
           + jnp.dot(ea_ref[...], we1_ref[...], preferred_element_type=jnp.float32)
           + b1_ref[...])
    hg = jax.nn.silu(pre[:, :H])
    ho = jax.nn.silu(pre[:, H:])
    gate = jax.nn.sigmoid(
        jnp.dot(hg, wg2_ref[...], preferred_element_type=jnp.float32) + bg2_ref[...])
    val = jax.nn.silu(
        jnp.dot(ho, wo2_ref[...], preferred_element_type=jnp.float32) + bo2_ref[...])
    msg_ref[...] = gate * val


def _final_body(x_ref, p_ref, wl_ref, bl_ref, out_ref):
    agg = p_ref[0] + p_ref[1]
    out_ref[...] = (x_ref[...]
                    + jnp.dot(agg, wl_ref[...], preferred_element_type=jnp.float32)
                    + bl_ref[...])


def kernel(x, edge_index, edge_attr, Wg1, bg1, Wg2, bg2, Wo1, bo1, Wo2, bo2, Wl, bl):
    f32 = jnp.float32
    w_src = jnp.concatenate([Wg1[:ND], Wo1[:ND]], axis=1)
    w_dst = jnp.concatenate([Wg1[ND:2 * ND], Wo1[ND:2 * ND]], axis=1)
    w_e = jnp.concatenate([Wg1[2 * ND:], Wo1[2 * ND:]], axis=1)
    b1 = jnp.concatenate([bg1, bo1]).reshape(1, 2 * H)
    bg2r = bg2.reshape(1, ND)
    bo2r = bo2.reshape(1, ND)
    blr = bl.reshape(1, ND)

    npad = EP - N_EDGES
    pad_gather = (jnp.arange(npad, dtype=jnp.int32) % N_NODES)
    pad_scatter = N_NODES + (jnp.arange(npad, dtype=jnp.int32) % (N_PAD - N_NODES))
    src = jnp.concatenate([edge_index[0], pad_gather]).reshape(NW, NCHUNK, CH)
    dst_g = jnp.concatenate([edge_index[1], pad_gather]).reshape(NW, NCHUNK, CH)
    dst_s = jnp.concatenate([edge_index[1], pad_scatter]).reshape(NW, NCHUNK, CH)
    ea_pad = jnp.concatenate(
        [edge_attr, jnp.zeros((npad, ED), dtype=f32)], axis=0)

    S, D = pl.pallas_call(
        _proj_body,
        out_shape=[jax.ShapeDtypeStruct((N_NODES, ND), f32),
                   jax.ShapeDtypeStruct((N_NODES, ND), f32)],
    )(x, w_src, w_dst)

    gather_add, scatter_sum = _sc_kernels()

    G = gather_add(S, D, src, dst_g).reshape(EP, ND)

    BE = 4096
    grid = EP // BE
    msg = pl.pallas_call(
        _msg_body,
        grid=(grid,),
        in_specs=[
            pl.BlockSpec((BE, ND), lambda i: (i, 0)),
            pl.BlockSpec((BE, ED), lambda i: (i, 0)),
            pl.BlockSpec((ED, 2 * H), lambda i: (0, 0)),
            pl.BlockSpec((1, 2 * H), lambda i: (0, 0)),
            pl.BlockSpec((H, ND), lambda i: (0, 0)),
            pl.BlockSpec((1, ND), lambda i: (0, 0)),
            pl.BlockSpec((H, ND), lambda i: (0, 0)),
            pl.BlockSpec((1, ND), lambda i: (0, 0)),
        ],
        out_specs=pl.BlockSpec((BE, ND), lambda i: (i, 0)),
        out_shape=jax.ShapeDtypeStruct((EP, ND), f32),
    )(G, ea_pad, w_e, b1, Wg2, bg2r, Wo2, bo2r)

    partials = scatter_sum(msg.reshape(NW, NCHUNK, CH, ND), dst_s)

    BN = 2000
    out = pl.pallas_call(
        _final_body,
        grid=(N_NODES // BN,),
        in_specs=[
            pl.BlockSpec((BN, ND), lambda i: (i, 0)),
            pl.BlockSpec((NC, BN, ND), lambda i: (0, i, 0)),
            pl.BlockSpec((ND, ND), lambda i: (0, 0)),
            pl.BlockSpec((1, ND), lambda i: (0, 0)),
        ],
        out_specs=pl.BlockSpec((BN, ND), lambda i: (i, 0)),
        out_shape=jax.ShapeDtypeStruct((N_NODES, ND), f32),
    )(x, partials, Wl, blr)
    return out


# speedup vs baseline: 1.1446x; 1.0217x over previous
import functools

import jax
import jax.numpy as jnp
from jax import lax
from jax.experimental import pallas as pl
from jax.experimental.pallas import tpu as pltpu
from jax.experimental.pallas import tpu_sc as plsc

N_NODES = 10000
N_EDGES = 320000
ND = 128
ED = 16
H = 64

NC = 2
NS = 16
L = 16
NW = NC * NS
CH = 128
NCHUNK = 80
EPW = CH * NCHUNK
EP = EPW * NW
NP = 4
NCH_P = NCHUNK // NP
EPP = EP // NP
N_PAD = 10240
RPT = N_PAD // NS
ZC = 128


def _gather_add_body(nchunk, s_hbm, d_hbm, sidx_hbm, didx_hbm, g_hbm,
                     sidx_v, didx_v, rs0, rd0, rs1, rd1,
                     sem_g0, sem_g1, sem_w0, sem_w1):
    wid = lax.axis_index("c") * NS + lax.axis_index("s")
    pltpu.sync_copy(sidx_hbm.at[wid], sidx_v)
    pltpu.sync_copy(didx_hbm.at[wid], didx_v)
    rs = (rs0, rs1)
    rd = (rd0, rd1)
    sg = (sem_g0, sem_g1)
    sw = (sem_w0, sem_w1)

    def g_start(j, b):
        pltpu.async_copy(s_hbm.at[sidx_v.at[j]], rs[b], sg[b])
        pltpu.async_copy(d_hbm.at[didx_v.at[j]], rd[b], sg[b])

    def g_wait(b):
        pltpu.make_async_copy(s_hbm.at[sidx_v.at[0]], rs[b], sg[b]).wait()
        pltpu.make_async_copy(d_hbm.at[didx_v.at[0]], rd[b], sg[b]).wait()

    def w_start(j, b):
        pltpu.async_copy(rs[b], g_hbm.at[wid, j], sw[b])

    def w_wait(b):
        pltpu.make_async_copy(rs[b], g_hbm.at[wid, 0], sw[b]).wait()

    g_start(0, 0)

    def outer(i, carry):
        for b in range(2):
            j = 2 * i + b
            nb = 1 - b

            @pl.when(j + 1 < nchunk)
            def _start_next():
                @pl.when(j >= 1)
                def _drain_write():
                    w_wait(nb)
                g_start(j + 1, nb)

            g_wait(b)

            def addrow(r, c2):
                for cc in range(ND // L):
                    sl = pl.ds(cc * L, L)
                    plsc.addupdate(rs[b].at[r, sl], rd[b][r, sl])
                return c2

            lax.fori_loop(0, CH, addrow, 0)
            w_start(j, b)
        return carry

    lax.fori_loop(0, nchunk // 2, outer, 0)
    w_wait(0)
    w_wait(1)


def _scatter_sum_body(msg0, msg1, msg2, msg3, didx_hbm, out_hbm,
                      didx_v, m0, m1, agg_sh,
                      sem_l0, sem_l1, sem_s0, sem_s1):
    cid = lax.axis_index("c")
    sid = lax.axis_index("s")
    wid = cid * NS + sid
    msgs = (msg0, msg1, msg2, msg3)
    m = (m0, m1)
    sl_ = (sem_l0, sem_l1)
    ss = (sem_s0, sem_s1)
    NB = 2

    def zrow(r, c2):
        for cc in range(ND // L):
            m0[r, pl.ds(cc * L, L)] = jnp.zeros((L,), jnp.float32)
        return c2

    lax.fori_loop(0, ZC, zrow, 0)
    for k in range(RPT // ZC):
        pltpu.sync_copy(m0, agg_sh.at[pl.ds(sid * RPT + k * ZC, ZC)])
    for p in range(NP):
        pltpu.sync_copy(didx_hbm.at[p, wid],
                        didx_v.at[pl.ds(p * NCH_P, NCH_P)])
    plsc.subcore_barrier()

    def s_start(j, b):
        pltpu.async_copy(m[b], agg_sh.at[didx_v.at[j]], ss[b], add=True)

    def s_wait(b):
        pltpu.make_async_copy(m[b], agg_sh.at[didx_v.at[0]], ss[b]).wait()

    for p in range(NP):
        mh = msgs[p]

        def l_start(c, b):
            pltpu.async_copy(mh.at[wid, c], m[b], sl_[b])

        def l_wait(b):
            pltpu.make_async_copy(mh.at[wid, 0], m[b], sl_[b]).wait()

        for b in range(NB):
            l_start(b, b)

        def outer(i, carry):
            for b in range(NB):
                c = NB * i + b
                l_wait(b)
                s_start(p * NCH_P + c, b)

                @pl.when(c + NB < NCH_P)
                def _start_next():
                    s_wait(b)
                    l_start(c + NB, b)
            return carry

        lax.fori_loop(0, NCH_P // NB, outer, 0)
        for b in range(NB):
            s_wait(b)
    plsc.subcore_barrier()
    pltpu.sync_copy(agg_sh.at[pl.ds(sid * RPT, RPT)],
                    out_hbm.at[cid, pl.ds(sid * RPT, RPT)])


@functools.lru_cache(maxsize=1)
def _sc_kernels():
    mesh = plsc.VectorSubcoreMesh(
        core_axis_name="c", subcore_axis_name="s", num_cores=NC, num_subcores=NS)
    gather_add = pl.kernel(
        functools.partial(_gather_add_body, NCH_P),
        out_type=jax.ShapeDtypeStruct((NW, NCH_P, CH, ND), jnp.float32),
        mesh=mesh,
        scratch_types=[
            pltpu.VMEM((NCH_P, CH), jnp.int32),
            pltpu.VMEM((NCH_P, CH), jnp.int32),
            pltpu.VMEM((CH, ND), jnp.float32),
            pltpu.VMEM((CH, ND), jnp.float32),
            pltpu.VMEM((CH, ND), jnp.float32),
            pltpu.VMEM((CH, ND), jnp.float32),
            pltpu.SemaphoreType.DMA,
            pltpu.SemaphoreType.DMA,
            pltpu.SemaphoreType.DMA,
            pltpu.SemaphoreType.DMA,
        ],
    )
    scatter_sum = pl.kernel(
        _scatter_sum_body,
        out_type=jax.ShapeDtypeStruct((NC, N_PAD, ND), jnp.float32),
        mesh=mesh,
        scratch_types=[
            pltpu.VMEM((NCHUNK, CH), jnp.int32),
            pltpu.VMEM((CH, ND), jnp.float32),
            pltpu.VMEM((CH, ND), jnp.float32),
            pltpu.VMEM_SHARED((N_PAD, ND), jnp.float32),
            pltpu.SemaphoreType.DMA,
            pltpu.SemaphoreType.DMA,
            pltpu.SemaphoreType.DMA,
            pltpu.SemaphoreType.DMA,
        ],
    )
    return gather_add, scatter_sum


def _proj_body(x_ref, ws_ref, wd_ref, s_ref, d_ref):
    xv = x_ref[...]
    s_ref[...] = jnp.dot(xv, ws_ref[...], preferred_element_type=jnp.float32)
    d_ref[...] = jnp.dot(xv, wd_ref[...], preferred_element_type=jnp.float32)


def _msg_body(g_ref, ea_ref, we1_ref, b1_ref, wg2_ref, bg2_ref,
              wo2_ref, bo2_ref, msg_ref):
    pre = (g_ref[...]
           + jnp.dot(ea_ref[...], we1_ref[...], preferred_element_type=jnp.float32)
           + b1_ref[...])
    hg = jax.nn.silu(pre[:, :H])
    ho = jax.nn.silu(pre[:, H:])
    gate = jax.nn.sigmoid(
        jnp.dot(hg, wg2_ref[...], preferred_element_type=jnp.float32) + bg2_ref[...])
    val = jax.nn.silu(
        jnp.dot(ho, wo2_ref[...], preferred_element_type=jnp.float32) + bo2_ref[...])
    msg_ref[...] = gate * val


def _final_body(x_ref, p_ref, wl_ref, bl_ref, out_ref):
    agg = p_ref[0] + p_ref[1]
    out_ref[...] = (x_ref[...]
                    + jnp.dot(agg, wl_ref[...], preferred_element_type=jnp.float32)
                    + bl_ref[...])


def kernel(x, edge_index, edge_attr, Wg1, bg1, Wg2, bg2, Wo1, bo1, Wo2, bo2, Wl, bl):
    f32 = jnp.float32
    w_src = jnp.concatenate([Wg1[:ND], Wo1[:ND]], axis=1)
    w_dst = jnp.concatenate([Wg1[ND:2 * ND], Wo1[ND:2 * ND]], axis=1)
    w_e = jnp.concatenate([Wg1[2 * ND:], Wo1[2 * ND:]], axis=1)
    b1 = jnp.concatenate([bg1, bo1]).reshape(1, 2 * H)
    bg2r = bg2.reshape(1, ND)
    bo2r = bo2.reshape(1, ND)
    blr = bl.reshape(1, ND)

    npad = EP - N_EDGES
    pad_gather = (jnp.arange(npad, dtype=jnp.int32) % N_NODES)
    pad_scatter = N_NODES + (jnp.arange(npad, dtype=jnp.int32) % (N_PAD - N_NODES))
    src = jnp.concatenate([edge_index[0], pad_gather]).reshape(NP, NW, NCH_P, CH)
    dst_g = jnp.concatenate([edge_index[1], pad_gather]).reshape(NP, NW, NCH_P, CH)
    dst_s = jnp.concatenate([edge_index[1], pad_scatter]).reshape(NP, NW, NCH_P, CH)
    ea_pad = jnp.concatenate(
        [edge_attr, jnp.zeros((npad, ED), dtype=f32)], axis=0)

    S, D = pl.pallas_call(
        _proj_body,
        out_shape=[jax.ShapeDtypeStruct((N_NODES, ND), f32),
                   jax.ShapeDtypeStruct((N_NODES, ND), f32)],
    )(x, w_src, w_dst)

    gather_add, scatter_sum = _sc_kernels()

    BE = 4096
    grid_p = EPP // BE
    msgs = []
    for p in range(NP):
        Gp = gather_add(S, D, src[p], dst_g[p]).reshape(EPP, ND)
        msg_p = pl.pallas_call(
            _msg_body,
            grid=(grid_p,),
            in_specs=[
                pl.BlockSpec((BE, ND), lambda i: (i, 0)),
                pl.BlockSpec((BE, ED), lambda i, p=p: (p * grid_p + i, 0)),
                pl.BlockSpec((ED, 2 * H), lambda i: (0, 0)),
                pl.BlockSpec((1, 2 * H), lambda i: (0, 0)),
                pl.BlockSpec((H, ND), lambda i: (0, 0)),
                pl.BlockSpec((1, ND), lambda i: (0, 0)),
                pl.BlockSpec((H, ND), lambda i: (0, 0)),
                pl.BlockSpec((1, ND), lambda i: (0, 0)),
            ],
            out_specs=pl.BlockSpec((BE, ND), lambda i: (i, 0)),
            out_shape=jax.ShapeDtypeStruct((EPP, ND), f32),
        )(Gp, ea_pad, w_e, b1, Wg2, bg2r, Wo2, bo2r)
        msgs.append(msg_p.reshape(NW, NCH_P, CH, ND))

    partials = scatter_sum(msgs[0], msgs[1], msgs[2], msgs[3], dst_s)

    BN = 2000
    out = pl.pallas_call(
        _final_body,
        grid=(N_NODES // BN,),
        in_specs=[
            pl.BlockSpec((BN, ND), lambda i: (i, 0)),
            pl.BlockSpec((NC, BN, ND), lambda i: (0, i, 0)),
            pl.BlockSpec((ND, ND), lambda i: (0, 0)),
            pl.BlockSpec((1, ND), lambda i: (0, 0)),
        ],
        out_specs=pl.BlockSpec((BN, ND), lambda i: (i, 0)),
        out_shape=jax.ShapeDtypeStruct((N_NODES, ND), f32),
    )(x, partials, Wl, blr)
    return out


# speedup vs baseline: 1.1615x; 1.0148x over previous
import functools

import jax
import jax.numpy as jnp
from jax import lax
from jax.experimental import pallas as pl
from jax.experimental.pallas import tpu as pltpu
from jax.experimental.pallas import tpu_sc as plsc

N_NODES = 10000
N_EDGES = 320000
ND = 128
ED = 16
H = 64

NC = 2
NS = 16
L = 16
NW = NC * NS
CH = 128
NCHUNK = 80
EPW = CH * NCHUNK
EP = EPW * NW
NP = 4
NCH_P = NCHUNK // NP
EPP = EP // NP
NCHT = EPP // NS // CH
N_PAD = 10240
STR = N_PAD // NS
RPT = N_PAD // NS
ZC = 128


def _cgather_body(s_hbm, d_hbm, sidx_hbm, didx_hbm, gs_hbm, gd_hbm,
                  idx_v, r0, r1, tab_sh,
                  sem_g0, sem_g1, sem_w0, sem_w1):
    cid = lax.axis_index("c")
    sid = lax.axis_index("s")
    st = pl.ds(sid * STR, STR)

    @pl.when(cid == 0)
    def _load_s():
        pltpu.sync_copy(s_hbm.at[st], tab_sh.at[st])
        pltpu.sync_copy(sidx_hbm.at[sid], idx_v)

    @pl.when(cid == 1)
    def _load_d():
        pltpu.sync_copy(d_hbm.at[st], tab_sh.at[st])
        pltpu.sync_copy(didx_hbm.at[sid], idx_v)

    plsc.subcore_barrier()

    r = (r0, r1)
    sg = (sem_g0, sem_g1)
    sw = (sem_w0, sem_w1)

    def ring(out_hbm):
        def g_start(j, b):
            pltpu.async_copy(tab_sh.at[idx_v.at[j]], r[b], sg[b])

        def g_wait(b):
            pltpu.make_async_copy(tab_sh.at[idx_v.at[0]], r[b], sg[b]).wait()

        def w_start(j, b):
            pltpu.async_copy(r[b], out_hbm.at[sid, j], sw[b])

        def w_wait(b):
            pltpu.make_async_copy(r[b], out_hbm.at[sid, 0], sw[b]).wait()

        g_start(0, 0)

        def outer(i, carry):
            for b in range(2):
                j = 2 * i + b
                nb = 1 - b

                @pl.when(j + 1 < NCHT)
                def _start_next():
                    @pl.when(j >= 1)
                    def _drain_write():
                        w_wait(nb)
                    g_start(j + 1, nb)

                g_wait(b)
                w_start(j, b)
            return carry

        lax.fori_loop(0, NCHT // 2, outer, 0)
        w_wait(0)
        w_wait(1)

    @pl.when(cid == 0)
    def _run_s():
        ring(gs_hbm)

    @pl.when(cid == 1)
    def _run_d():
        ring(gd_hbm)


def _scatter_sum_body(msg0, msg1, msg2, msg3, didx_hbm, out_hbm,
                      didx_v, m0, m1, agg_sh,
                      sem_l0, sem_l1, sem_s0, sem_s1):
    cid = lax.axis_index("c")
    sid = lax.axis_index("s")
    wid = cid * NS + sid
    msgs = (msg0, msg1, msg2, msg3)
    m = (m0, m1)
    sl_ = (sem_l0, sem_l1)
    ss = (sem_s0, sem_s1)
    NB = 2

    def zrow(r, c2):
        for cc in range(ND // L):
            m0[r, pl.ds(cc * L, L)] = jnp.zeros((L,), jnp.float32)
        return c2

    lax.fori_loop(0, ZC, zrow, 0)
    for k in range(RPT // ZC):
        pltpu.sync_copy(m0, agg_sh.at[pl.ds(sid * RPT + k * ZC, ZC)])
    for p in range(NP):
        pltpu.sync_copy(didx_hbm.at[p, wid],
                        didx_v.at[pl.ds(p * NCH_P, NCH_P)])
    plsc.subcore_barrier()

    def s_start(j, b):
        pltpu.async_copy(m[b], agg_sh.at[didx_v.at[j]], ss[b], add=True)

    def s_wait(b):
        pltpu.make_async_copy(m[b], agg_sh.at[didx_v.at[0]], ss[b]).wait()

    for p in range(NP):
        mh = msgs[p]

        def l_start(c, b):
            pltpu.async_copy(mh.at[wid, c], m[b], sl_[b])

        def l_wait(b):
            pltpu.make_async_copy(mh.at[wid, 0], m[b], sl_[b]).wait()

        for b in range(NB):
            l_start(b, b)

        def outer(i, carry):
            for b in range(NB):
                c = NB * i + b
                l_wait(b)
                s_start(p * NCH_P + c, b)

                @pl.when(c + NB < NCH_P)
                def _start_next():
                    s_wait(b)
                    l_start(c + NB, b)
            return carry

        lax.fori_loop(0, NCH_P // NB, outer, 0)
        for b in range(NB):
            s_wait(b)
    plsc.subcore_barrier()
    pltpu.sync_copy(agg_sh.at[pl.ds(sid * RPT, RPT)],
                    out_hbm.at[cid, pl.ds(sid * RPT, RPT)])


@functools.lru_cache(maxsize=1)
def _sc_kernels():
    mesh = plsc.VectorSubcoreMesh(
        core_axis_name="c", subcore_axis_name="s", num_cores=NC, num_subcores=NS)
    cgather = pl.kernel(
        _cgather_body,
        out_type=[jax.ShapeDtypeStruct((NS, NCHT, CH, ND), jnp.float32),
                  jax.ShapeDtypeStruct((NS, NCHT, CH, ND), jnp.float32)],
        mesh=mesh,
        scratch_types=[
            pltpu.VMEM((NCHT, CH), jnp.int32),
            pltpu.VMEM((CH, ND), jnp.float32),
            pltpu.VMEM((CH, ND), jnp.float32),
            pltpu.VMEM_SHARED((N_PAD, ND), jnp.float32),
            pltpu.SemaphoreType.DMA,
            pltpu.SemaphoreType.DMA,
            pltpu.SemaphoreType.DMA,
            pltpu.SemaphoreType.DMA,
        ],
    )
    scatter_sum = pl.kernel(
        _scatter_sum_body,
        out_type=jax.ShapeDtypeStruct((NC, N_PAD, ND), jnp.float32),
        mesh=mesh,
        scratch_types=[
            pltpu.VMEM((NCHUNK, CH), jnp.int32),
            pltpu.VMEM((CH, ND), jnp.float32),
            pltpu.VMEM((CH, ND), jnp.float32),
            pltpu.VMEM_SHARED((N_PAD, ND), jnp.float32),
            pltpu.SemaphoreType.DMA,
            pltpu.SemaphoreType.DMA,
            pltpu.SemaphoreType.DMA,
            pltpu.SemaphoreType.DMA,
        ],
    )
    return cgather, scatter_sum


def _proj_body(x_ref, ws_ref, wd_ref, s_ref, d_ref):
    xv = x_ref[...]
    s_ref[...] = jnp.dot(xv, ws_ref[...], preferred_element_type=jnp.float32)
    d_ref[...] = jnp.dot(xv, wd_ref[...], preferred_element_type=jnp.float32)


def _msg_body(gs_ref, gd_ref, ea_ref, we1_ref, b1_ref, wg2_ref, bg2_ref,
              wo2_ref, bo2_ref, msg_ref):
    pre = (gs_ref[...] + gd_ref[...]
           + jnp.dot(ea_ref[...], we1_ref[...], preferred_element_type=jnp.float32)
           + b1_ref[...])
    hg = jax.nn.silu(pre[:, :H])
    ho = jax.nn.silu(pre[:, H:])
    gate = jax.nn.sigmoid(
        jnp.dot(hg, wg2_ref[...], preferred_element_type=jnp.float32) + bg2_ref[...])
    val = jax.nn.silu(
        jnp.dot(ho, wo2_ref[...], preferred_element_type=jnp.float32) + bo2_ref[...])
    msg_ref[...] = gate * val


def _final_body(x_ref, p_ref, wl_ref, bl_ref, out_ref):
    agg = p_ref[0] + p_ref[1]
    out_ref[...] = (x_ref[...]
                    + jnp.dot(agg, wl_ref[...], preferred_element_type=jnp.float32)
                    + bl_ref[...])


def kernel(x, edge_index, edge_attr, Wg1, bg1, Wg2, bg2, Wo1, bo1, Wo2, bo2, Wl, bl):
    f32 = jnp.float32
    w_src = jnp.concatenate([Wg1[:ND], Wo1[:ND]], axis=1)
    w_dst = jnp.concatenate([Wg1[ND:2 * ND], Wo1[ND:2 * ND]], axis=1)
    w_e = jnp.concatenate([Wg1[2 * ND:], Wo1[2 * ND:]], axis=1)
    b1 = jnp.concatenate([bg1, bo1]).reshape(1, 2 * H)
    bg2r = bg2.reshape(1, ND)
    bo2r = bo2.reshape(1, ND)
    blr = bl.reshape(1, ND)

    npad = EP - N_EDGES
    pad_gather = (jnp.arange(npad, dtype=jnp.int32) % N_NODES)
    pad_scatter = N_NODES + (jnp.arange(npad, dtype=jnp.int32) % (N_PAD - N_NODES))
    src = jnp.concatenate([edge_index[0], pad_gather]).reshape(NP, NS, NCHT, CH)
    dst_g = jnp.concatenate([edge_index[1], pad_gather]).reshape(NP, NS, NCHT, CH)
    dst_s = jnp.concatenate([edge_index[1], pad_scatter]).reshape(NP, NW, NCH_P, CH)
    ea_pad = jnp.concatenate(
        [edge_attr, jnp.zeros((npad, ED), dtype=f32)], axis=0)

    xp = jnp.pad(x, ((0, N_PAD - N_NODES), (0, 0)))
    S, D = pl.pallas_call(
        _proj_body,
        out_shape=[jax.ShapeDtypeStruct((N_PAD, ND), f32),
                   jax.ShapeDtypeStruct((N_PAD, ND), f32)],
    )(xp, w_src, w_dst)

    cgather, scatter_sum = _sc_kernels()

    BE = 4096
    grid_p = EPP // BE
    msgs = []
    for p in range(NP):
        GSp, GDp = cgather(S, D, src[p], dst_g[p])
        msg_p = pl.pallas_call(
            _msg_body,
            grid=(grid_p,),
            in_specs=[
                pl.BlockSpec((BE, ND), lambda i: (i, 0)),
                pl.BlockSpec((BE, ND), lambda i: (i, 0)),
                pl.BlockSpec((BE, ED), lambda i, p=p: (p * grid_p + i, 0)),
                pl.BlockSpec((ED, 2 * H), lambda i: (0, 0)),
                pl.BlockSpec((1, 2 * H), lambda i: (0, 0)),
                pl.BlockSpec((H, ND), lambda i: (0, 0)),
                pl.BlockSpec((1, ND), lambda i: (0, 0)),
                pl.BlockSpec((H, ND), lambda i: (0, 0)),
                pl.BlockSpec((1, ND), lambda i: (0, 0)),
            ],
            out_specs=pl.BlockSpec((BE, ND), lambda i: (i, 0)),
            out_shape=jax.ShapeDtypeStruct((EPP, ND), f32),
        )(GSp.reshape(EPP, ND), GDp.reshape(EPP, ND),
          ea_pad, w_e, b1, Wg2, bg2r, Wo2, bo2r)
        msgs.append(msg_p.reshape(NW, NCH_P, CH, ND))

    partials = scatter_sum(msgs[0], msgs[1], msgs[2], msgs[3], dst_s)

    BN = 2000
    out = pl.pallas_call(
        _final_body,
        grid=(N_NODES // BN,),
        in_specs=[
            pl.BlockSpec((BN, ND), lambda i: (i, 0)),
            pl.BlockSpec((NC, BN, ND), lambda i: (0, i, 0)),
            pl.BlockSpec((ND, ND), lambda i: (0, 0)),
            pl.BlockSpec((1, ND), lambda i: (0, 0)),
        ],
        out_specs=pl.BlockSpec((BN, ND), lambda i: (i, 0)),
        out_shape=jax.ShapeDtypeStruct((N_NODES, ND), f32),
    )(x, partials, Wl, blr)
    return out


# speedup vs baseline: 1.1774x; 1.0137x over previous
import functools

import jax
import jax.numpy as jnp
from jax import lax
from jax.experimental import pallas as pl
from jax.experimental.pallas import tpu as pltpu
from jax.experimental.pallas import tpu_sc as plsc

N_NODES = 10000
N_EDGES = 320000
ND = 128
ED = 16
H = 64

NC = 2
NS = 16
L = 16
NW = NC * NS
CH = 128
NCHUNK = 80
EPW = CH * NCHUNK
EP = EPW * NW
NP = 4
NCH_P = NCHUNK // NP
EPP = EP // NP
EPT = EPP // NS
EPPW = EPP // NW
GCH = 64
GNB = 4
GNCH = EPT // GCH
SCH = 64
SNB = 4
SNCH = EPPW // SCH
N_PAD = 10240
STR = N_PAD // NS
RPT = N_PAD // NS
ZC = 128


def _cgather_body(s_hbm, d_hbm, sidx_hbm, didx_hbm, gs_hbm, gd_hbm,
                  idx_v, r0, r1, r2, r3, tab_sh,
                  sem_g0, sem_g1, sem_g2, sem_g3,
                  sem_w0, sem_w1, sem_w2, sem_w3):
    cid = lax.axis_index("c")
    sid = lax.axis_index("s")
    st = pl.ds(sid * STR, STR)

    @pl.when(cid == 0)
    def _load_s():
        pltpu.sync_copy(s_hbm.at[st], tab_sh.at[st])
        pltpu.sync_copy(sidx_hbm.at[sid], idx_v)

    @pl.when(cid == 1)
    def _load_d():
        pltpu.sync_copy(d_hbm.at[st], tab_sh.at[st])
        pltpu.sync_copy(didx_hbm.at[sid], idx_v)

    plsc.subcore_barrier()

    r = (r0, r1, r2, r3)
    sg = (sem_g0, sem_g1, sem_g2, sem_g3)
    sw = (sem_w0, sem_w1, sem_w2, sem_w3)

    def ring(out_hbm):
        def g_start(j, b):
            pltpu.async_copy(tab_sh.at[idx_v.at[pl.ds(j * GCH, GCH)]],
                             r[b], sg[b])

        def g_wait(b):
            pltpu.make_async_copy(
                tab_sh.at[idx_v.at[pl.ds(0, GCH)]], r[b], sg[b]).wait()

        def w_start(j, b):
            pltpu.async_copy(r[b], out_hbm.at[sid, pl.ds(j * GCH, GCH)], sw[b])

        def w_wait(b):
            pltpu.make_async_copy(
                r[b], out_hbm.at[sid, pl.ds(0, GCH)], sw[b]).wait()

        g_start(0, 0)

        def outer(i, carry):
            for b in range(GNB):
                j = GNB * i + b
                nb = (b + 1) % GNB

                @pl.when(j + 1 < GNCH)
                def _start_next():
                    @pl.when(j + 1 >= GNB)
                    def _drain_write():
                        w_wait(nb)
                    g_start(j + 1, nb)

                g_wait(b)
                w_start(j, b)
            return carry

        lax.fori_loop(0, GNCH // GNB, outer, 0)
        for b in range(GNB):
            w_wait(b)

    @pl.when(cid == 0)
    def _run_s():
        ring(gs_hbm)

    @pl.when(cid == 1)
    def _run_d():
        ring(gd_hbm)


def _scatter_sum_body(msg0, msg1, msg2, msg3, didx_hbm, out_hbm,
                      didx_v, m0, m1, m2, m3, agg_sh,
                      sem_l0, sem_l1, sem_l2, sem_l3,
                      sem_s0, sem_s1, sem_s2, sem_s3):
    cid = lax.axis_index("c")
    sid = lax.axis_index("s")
    wid = cid * NS + sid
    msgs = (msg0, msg1, msg2, msg3)
    m = (m0, m1, m2, m3)
    sl_ = (sem_l0, sem_l1, sem_l2, sem_l3)
    ss = (sem_s0, sem_s1, sem_s2, sem_s3)
    NB = SNB

    def zrow(r, c2):
        for cc in range(ND // L):
            m0[r, pl.ds(cc * L, L)] = jnp.zeros((L,), jnp.float32)
        return c2

    lax.fori_loop(0, SCH, zrow, 0)
    for k in range(RPT // SCH):
        pltpu.sync_copy(m0, agg_sh.at[pl.ds(sid * RPT + k * SCH, SCH)])
    for p in range(NP):
        pltpu.sync_copy(didx_hbm.at[p, wid],
                        didx_v.at[pl.ds(p * EPPW, EPPW)])
    plsc.subcore_barrier()

    def s_start(j, b):
        pltpu.async_copy(m[b], agg_sh.at[didx_v.at[pl.ds(j * SCH, SCH)]],
                         ss[b], add=True)

    def s_wait(b):
        pltpu.make_async_copy(m[b], agg_sh.at[didx_v.at[pl.ds(0, SCH)]],
                              ss[b]).wait()

    for p in range(NP):
        mh = msgs[p]

        def l_start(c, b):
            pltpu.async_copy(mh.at[wid, pl.ds(c * SCH, SCH)], m[b], sl_[b])

        def l_wait(b):
            pltpu.make_async_copy(mh.at[wid, pl.ds(0, SCH)], m[b], sl_[b]).wait()

        for b in range(NB):
            l_start(b, b)

        def outer(i, carry):
            for b in range(NB):
                c = NB * i + b
                l_wait(b)
                s_start(p * SNCH + c, b)

                @pl.when(c + NB < SNCH)
                def _start_next():
                    s_wait(b)
                    l_start(c + NB, b)
            return carry

        lax.fori_loop(0, SNCH // NB, outer, 0)
        for b in range(NB):
            s_wait(b)
    plsc.subcore_barrier()
    pltpu.sync_copy(agg_sh.at[pl.ds(sid * RPT, RPT)],
                    out_hbm.at[cid, pl.ds(sid * RPT, RPT)])


@functools.lru_cache(maxsize=1)
def _sc_kernels():
    mesh = plsc.VectorSubcoreMesh(
        core_axis_name="c", subcore_axis_name="s", num_cores=NC, num_subcores=NS)
    cgather = pl.kernel(
        _cgather_body,
        out_type=[jax.ShapeDtypeStruct((NS, EPT, ND), jnp.float32),
                  jax.ShapeDtypeStruct((NS, EPT, ND), jnp.float32)],
        mesh=mesh,
        scratch_types=(
            [pltpu.VMEM((EPT,), jnp.int32)]
            + [pltpu.VMEM((GCH, ND), jnp.float32)] * GNB
            + [pltpu.VMEM_SHARED((N_PAD, ND), jnp.float32)]
            + [pltpu.SemaphoreType.DMA] * (2 * GNB)
        ),
    )
    scatter_sum = pl.kernel(
        _scatter_sum_body,
        out_type=jax.ShapeDtypeStruct((NC, N_PAD, ND), jnp.float32),
        mesh=mesh,
        scratch_types=(
            [pltpu.VMEM((EPW,), jnp.int32)]
            + [pltpu.VMEM((SCH, ND), jnp.float32)] * SNB
            + [pltpu.VMEM_SHARED((N_PAD, ND), jnp.float32)]
            + [pltpu.SemaphoreType.DMA] * (2 * SNB)
        ),
    )
    return cgather, scatter_sum


def _proj_body(x_ref, ws_ref, wd_ref, s_ref, d_ref):
    xv = x_ref[...]
    s_ref[...] = jnp.dot(xv, ws_ref[...], preferred_element_type=jnp.float32)
    d_ref[...] = jnp.dot(xv, wd_ref[...], preferred_element_type=jnp.float32)


def _msg_body(gs_ref, gd_ref, ea_ref, we1_ref, b1_ref, wg2_ref, bg2_ref,
              wo2_ref, bo2_ref, msg_ref):
    pre = (gs_ref[...] + gd_ref[...]
           + jnp.dot(ea_ref[...], we1_ref[...], preferred_element_type=jnp.float32)
           + b1_ref[...])
    hg = jax.nn.silu(pre[:, :H])
    ho = jax.nn.silu(pre[:, H:])
    gate = jax.nn.sigmoid(
        jnp.dot(hg, wg2_ref[...], preferred_element_type=jnp.float32) + bg2_ref[...])
    val = jax.nn.silu(
        jnp.dot(ho, wo2_ref[...], preferred_element_type=jnp.float32) + bo2_ref[...])
    msg_ref[...] = gate * val


def _final_body(x_ref, p_ref, wl_ref, bl_ref, out_ref):
    agg = p_ref[0] + p_ref[1]
    out_ref[...] = (x_ref[...]
                    + jnp.dot(agg, wl_ref[...], preferred_element_type=jnp.float32)
                    + bl_ref[...])


def kernel(x, edge_index, edge_attr, Wg1, bg1, Wg2, bg2, Wo1, bo1, Wo2, bo2, Wl, bl):
    f32 = jnp.float32
    w_src = jnp.concatenate([Wg1[:ND], Wo1[:ND]], axis=1)
    w_dst = jnp.concatenate([Wg1[ND:2 * ND], Wo1[ND:2 * ND]], axis=1)
    w_e = jnp.concatenate([Wg1[2 * ND:], Wo1[2 * ND:]], axis=1)
    b1 = jnp.concatenate([bg1, bo1]).reshape(1, 2 * H)
    bg2r = bg2.reshape(1, ND)
    bo2r = bo2.reshape(1, ND)
    blr = bl.reshape(1, ND)

    npad = EP - N_EDGES
    pad_gather = (jnp.arange(npad, dtype=jnp.int32) % N_NODES)
    pad_scatter = N_NODES + (jnp.arange(npad, dtype=jnp.int32) % (N_PAD - N_NODES))
    src = jnp.concatenate([edge_index[0], pad_gather]).reshape(NP, NS, EPT)
    dst_g = jnp.concatenate([edge_index[1], pad_gather]).reshape(NP, NS, EPT)
    dst_s = jnp.concatenate([edge_index[1], pad_scatter]).reshape(NP, NW, EPPW)
    ea_pad = jnp.concatenate(
        [edge_attr, jnp.zeros((npad, ED), dtype=f32)], axis=0)

    xp = jnp.pad(x, ((0, N_PAD - N_NODES), (0, 0)))
    S, D = pl.pallas_call(
        _proj_body,
        out_shape=[jax.ShapeDtypeStruct((N_PAD, ND), f32),
                   jax.ShapeDtypeStruct((N_PAD, ND), f32)],
    )(xp, w_src, w_dst)

    cgather, scatter_sum = _sc_kernels()

    BE = 4096
    grid_p = EPP // BE
    msgs = []
    for p in range(NP):
        GSp, GDp = cgather(S, D, src[p], dst_g[p])
        msg_p = pl.pallas_call(
            _msg_body,
            grid=(grid_p,),
            in_specs=[
                pl.BlockSpec((BE, ND), lambda i: (i, 0)),
                pl.BlockSpec((BE, ND), lambda i: (i, 0)),
                pl.BlockSpec((BE, ED), lambda i, p=p: (p * grid_p + i, 0)),
                pl.BlockSpec((ED, 2 * H), lambda i: (0, 0)),
                pl.BlockSpec((1, 2 * H), lambda i: (0, 0)),
                pl.BlockSpec((H, ND), lambda i: (0, 0)),
                pl.BlockSpec((1, ND), lambda i: (0, 0)),
                pl.BlockSpec((H, ND), lambda i: (0, 0)),
                pl.BlockSpec((1, ND), lambda i: (0, 0)),
            ],
            out_specs=pl.BlockSpec((BE, ND), lambda i: (i, 0)),
            out_shape=jax.ShapeDtypeStruct((EPP, ND), f32),
        )(GSp.reshape(EPP, ND), GDp.reshape(EPP, ND),
          ea_pad, w_e, b1, Wg2, bg2r, Wo2, bo2r)
        msgs.append(msg_p.reshape(NW, EPPW, ND))

    partials = scatter_sum(msgs[0], msgs[1], msgs[2], msgs[3], dst_s)

    BN = 2000
    out = pl.pallas_call(
        _final_body,
        grid=(N_NODES // BN,),
        in_specs=[
            pl.BlockSpec((BN, ND), lambda i: (i, 0)),
            pl.BlockSpec((NC, BN, ND), lambda i: (0, i, 0)),
            pl.BlockSpec((ND, ND), lambda i: (0, 0)),
            pl.BlockSpec((1, ND), lambda i: (0, 0)),
        ],
        out_specs=pl.BlockSpec((BN, ND), lambda i: (i, 0)),
        out_shape=jax.ShapeDtypeStruct((N_NODES, ND), f32),
    )(x, partials, Wl, blr)
    return out
